# baseline (speedup 1.0000x reference)
import functools

import jax
import jax.numpy as jnp
from jax import lax
from jax.experimental import pallas as pl
from jax.experimental.pallas import tpu as pltpu
from jax.experimental.pallas import tpu_sc as plsc

_NC = 2
_NS = 16
_NW = _NC * _NS
_L = 16


def kernel(camera_idx, rotations, translations):
    B, = camera_idx.shape
    N, D = rotations.shape
    per_w = B // _NW
    n_vec = per_w // _L

    rot_flat = rotations.reshape(-1)
    trans_flat = translations.reshape(-1)

    mesh = plsc.VectorSubcoreMesh(core_axis_name="c", subcore_axis_name="s")

    @functools.partial(
        pl.kernel,
        out_type=(
            jax.ShapeDtypeStruct((B * D,), jnp.float32),
            jax.ShapeDtypeStruct((B * D,), jnp.float32),
        ),
        mesh=mesh,
        scratch_types=[
            pltpu.VMEM((per_w,), jnp.int32),
            pltpu.VMEM((per_w * D,), jnp.int32),
            pltpu.VMEM((per_w * D,), jnp.float32),
            pltpu.VMEM((per_w * D,), jnp.float32),
            pltpu.SemaphoreType.DMA,
            pltpu.SemaphoreType.DMA,
        ],
        compiler_params=pltpu.CompilerParams(
            use_tc_tiling_on_sc=False, needs_layout_passes=False),
    )
    def _gather(idx_hbm, rot_hbm, trans_hbm, rot_out, trans_out,
                idx_v, eidx_v, rot_v, trans_v, sem_r, sem_t):
        wid = lax.axis_index("s") * _NC + lax.axis_index("c")
        base = wid * per_w

        pltpu.sync_copy(idx_hbm.at[pl.ds(base, per_w)], idx_v)

        lane = lax.iota(jnp.int32, _L)
        for k in range(n_vec):
            v = idx_v[pl.ds(k * _L, _L)] * D
            pos = lane * D + (k * _L * D)
            for c in range(D):
                plsc.store_scatter(eidx_v, [pos + c], v + c)

        cr = pltpu.async_copy(rot_hbm.at[eidx_v], rot_v, sem_r)
        ct = pltpu.async_copy(trans_hbm.at[eidx_v], trans_v, sem_t)
        cr.wait()
        ct.wait()

        pltpu.sync_copy(rot_v, rot_out.at[pl.ds(base * D, per_w * D)])
        pltpu.sync_copy(trans_v, trans_out.at[pl.ds(base * D, per_w * D)])

    rot_flat_out, trans_flat_out = _gather(camera_idx, rot_flat, trans_flat)
    return (rot_flat_out.reshape(B, D), trans_flat_out.reshape(B, D))


# speedup vs baseline: 5.6958x; 5.6958x over previous
import functools

import jax
import jax.numpy as jnp
from jax import lax
from jax.experimental import pallas as pl
from jax.experimental.pallas import tpu as pltpu
from jax.experimental.pallas import tpu_sc as plsc

_NC = 2
_NS = 16
_NW = _NC * _NS
_L = 16


def kernel(camera_idx, rotations, translations):
    B, = camera_idx.shape
    N, D = rotations.shape
    per_w = B // _NW
    n_vec = per_w // _L

    rot_flat = rotations.T.reshape(-1)
    trans_flat = translations.T.reshape(-1)

    mesh = plsc.VectorSubcoreMesh(core_axis_name="c", subcore_axis_name="s")

    @functools.partial(
        pl.kernel,
        out_type=(
            jax.ShapeDtypeStruct((D * B,), jnp.float32),
            jax.ShapeDtypeStruct((D * B,), jnp.float32),
        ),
        mesh=mesh,
        scratch_types=[
            pltpu.VMEM((D, per_w), jnp.int32),
            pltpu.VMEM((D, per_w), jnp.float32),
            pltpu.VMEM((D, per_w), jnp.float32),
            pltpu.SemaphoreType.DMA,
            pltpu.SemaphoreType.DMA,
        ],
        compiler_params=pltpu.CompilerParams(use_tc_tiling_on_sc=False),
    )
    def _gather(idx_hbm, rot_hbm, trans_hbm, rot_out, trans_out,
                idx_v, rot_v, trans_v, sem_r, sem_t):
        wid = lax.axis_index("s") * _NC + lax.axis_index("c")
        base = wid * per_w

        pltpu.sync_copy(idx_hbm.at[pl.ds(base, per_w)], idx_v.at[0])
        for k in range(n_vec):
            v = idx_v[0, pl.ds(k * _L, _L)]
            for c in range(1, D):
                idx_v[c, pl.ds(k * _L, _L)] = v + (c * N)

        copies = []
        for c in range(D):
            copies.append(
                pltpu.async_copy(rot_hbm.at[idx_v.at[c]], rot_v.at[c], sem_r))
            copies.append(
                pltpu.async_copy(trans_hbm.at[idx_v.at[c]], trans_v.at[c],
                                 sem_t))
        for cp in copies:
            cp.wait()

        for c in range(D):
            pltpu.sync_copy(rot_v.at[c],
                            rot_out.at[pl.ds(c * B + base, per_w)])
            pltpu.sync_copy(trans_v.at[c],
                            trans_out.at[pl.ds(c * B + base, per_w)])

    rot_soa, trans_soa = _gather(camera_idx, rot_flat, trans_flat)
    return (rot_soa.reshape(D, B).T, trans_soa.reshape(D, B).T)
